# baseline (speedup 1.0000x reference)
import functools

import jax
import jax.numpy as jnp
from jax import lax
from jax.experimental import pallas as pl
from jax.experimental.pallas import tpu as pltpu
from jax.experimental.pallas import tpu_sc as plsc

_INFO = plsc.get_sparse_core_info()
_NC, _NS = _INFO.num_cores, _INFO.num_subcores
_NW = _NC * _NS
_L = _INFO.num_lanes

_NB = 4


def _embed_lookup(idx3, table, b, s, d):
    mesh = plsc.VectorSubcoreMesh(core_axis_name="c", subcore_axis_name="s")
    bl = b // _NW
    dt = d // 8

    @functools.partial(
        pl.kernel,
        out_type=jax.ShapeDtypeStruct((s, dt, _NW, 8, bl), jnp.float32),
        mesh=mesh,
        compiler_params=pltpu.CompilerParams(use_tc_tiling_on_sc=False, needs_layout_passes=False),
        scratch_types=[
            pltpu.VMEM((s, bl), jnp.int32),
            pltpu.VMEM((_NB, bl, d), jnp.float32),
            pltpu.VMEM((_NB, dt, 8, bl), jnp.float32),
            pltpu.SemaphoreType.DMA((_NB,)),
            pltpu.SemaphoreType.DMA((_NB,)),
        ],
    )
    def k(idx_hbm, table_hbm, out_hbm, idx_v, rows_v, rt_v, gsem, wsem):
        wid = lax.axis_index("s") * _NC + lax.axis_index("c")
        pltpu.sync_copy(idx_hbm.at[wid], idx_v)

        def gather(si, buf):
            return pltpu.make_async_copy(
                table_hbm.at[idx_v.at[si]], rows_v.at[buf], gsem.at[buf])

        def write(si, buf):
            return pltpu.make_async_copy(
                rt_v.at[buf], out_hbm.at[si, pl.ds(0, dt), wid], wsem.at[buf])

        def transpose(buf):
            buf_ids = jnp.full((_L,), buf, jnp.int32)

            def tbody(g, carry):
                row_ids = lax.iota(jnp.int32, _L) + g * _L
                for di in range(d):
                    col_ids = jnp.full((_L,), di, jnp.int32)
                    v = plsc.load_gather(rows_v, [buf_ids, row_ids, col_ids])
                    rt_v[buf, di // 8, di % 8, pl.ds(g * _L, _L)] = v
                return carry

            lax.fori_loop(0, bl // _L, tbody, 0)

        for buf in range(_NB):
            gather(buf, buf).start()

        n_groups = s // _NB

        def body(g, carry):
            for buf in range(_NB):
                si = g * _NB + buf
                gather(si, buf).wait()

                @pl.when(g > 0)
                def _():
                    write(si - _NB, buf).wait()

                transpose(buf)
                write(si, buf).start()
                gather(si + _NB, buf).start()
            return carry

        lax.fori_loop(0, n_groups - 1, body, 0)

        g = n_groups - 1
        for buf in range(_NB):
            si = g * _NB + buf
            gather(si, buf).wait()
            write(si - _NB, buf).wait()
            transpose(buf)
            write(si, buf).start()
        for buf in range(_NB):
            write(g * _NB + buf, buf).wait()

    return k(idx3, table)


def kernel(input_, weight):
    b, s = input_.shape
    d = weight.shape[1]
    bl = b // _NW
    assert b % _NW == 0 and s % _NB == 0 and d % 8 == 0 and bl % _L == 0
    idx3 = input_.reshape(_NW, bl, s).transpose(0, 2, 1).astype(jnp.int32)
    out5 = _embed_lookup(idx3, weight, b, s, d)
    return out5.transpose(2, 4, 0, 1, 3).reshape(b, s, d)


# speedup vs baseline: 2.2188x; 2.2188x over previous
import functools

import jax
import jax.numpy as jnp
from jax import lax
from jax.experimental import pallas as pl
from jax.experimental.pallas import tpu as pltpu
from jax.experimental.pallas import tpu_sc as plsc

_INFO = plsc.get_sparse_core_info()
_NC, _NS = _INFO.num_cores, _INFO.num_subcores
_NW = _NC * _NS
_L = _INFO.num_lanes

_NB = 5


def _embed_lookup(idx3, table, b, s, d):
    mesh = plsc.VectorSubcoreMesh(core_axis_name="c", subcore_axis_name="s")
    bl = b // _NW
    dt = d // 8

    @functools.partial(
        pl.kernel,
        out_type=jax.ShapeDtypeStruct((s, dt, _NW, 8, bl), jnp.float32),
        mesh=mesh,
        compiler_params=pltpu.CompilerParams(use_tc_tiling_on_sc=False, needs_layout_passes=False),
        scratch_types=[
            pltpu.VMEM((s, bl), jnp.int32),
            pltpu.VMEM((_NB, bl, d), jnp.float32),
            pltpu.VMEM((_NB, dt, 8, bl + 1), jnp.float32),
            pltpu.SemaphoreType.DMA((_NB,)),
            pltpu.SemaphoreType.DMA((_NB,)),
        ],
    )
    def k(idx_hbm, table_hbm, out_hbm, idx_v, rows_v, rt_v, gsem, wsem):
        wid = lax.axis_index("s") * _NC + lax.axis_index("c")
        pltpu.sync_copy(idx_hbm.at[wid], idx_v)

        def gather(si, buf):
            return pltpu.make_async_copy(
                table_hbm.at[idx_v.at[si]], rows_v.at[buf], gsem.at[buf])

        def write(si, buf):
            return pltpu.make_async_copy(
                rt_v.at[buf, pl.ds(0, dt), pl.ds(0, 8), pl.ds(0, bl)],
                out_hbm.at[si, pl.ds(0, dt), wid], wsem.at[buf])

        iota16 = lax.iota(jnp.int32, _L)
        ones16 = jnp.full((_L,), 1, jnp.int32)

        def transpose(buf):
            buf_ids = jnp.full((_L,), buf, jnp.int32)

            dgs = []
            for g in range(d // _L):
                dg = iota16 + g * _L
                dgs.append((dg // 8, dg % 8))

            @plsc.parallel_loop(0, bl // 4, unroll=2)
            def tbody(r4):
                vs = []
                for k in range(4):
                    r = r4 * 4 + k
                    for g in range(d // _L):
                        vs.append(rows_v[buf, r, pl.ds(g * _L, _L)])
                for k in range(4):
                    r = r4 * 4 + k
                    bl_ids = ones16 * r
                    for g in range(d // _L):
                        plsc.store_scatter(
                            rt_v, [buf_ids, dgs[g][0], dgs[g][1], bl_ids],
                            vs[k * (d // _L) + g])

        for buf in range(_NB):
            gather(buf, buf).start()

        n_groups = s // _NB

        def body(g, carry):
            for buf in range(_NB):
                si = g * _NB + buf
                gather(si, buf).wait()

                @pl.when(g > 0)
                def _():
                    write(si - _NB, buf).wait()

                transpose(buf)
                write(si, buf).start()
                gather(si + _NB, buf).start()
            return carry

        lax.fori_loop(0, n_groups - 1, body, 0)

        g = n_groups - 1
        for buf in range(_NB):
            si = g * _NB + buf
            gather(si, buf).wait()
            write(si - _NB, buf).wait()
            transpose(buf)
            write(si, buf).start()
        for buf in range(_NB):
            write(g * _NB + buf, buf).wait()

    return k(idx3, table)


def kernel(input_, weight):
    b, s = input_.shape
    d = weight.shape[1]
    bl = b // _NW
    assert b % _NW == 0 and s % _NB == 0 and d % 8 == 0 and bl % _L == 0
    idx3 = input_.reshape(_NW, bl, s).transpose(0, 2, 1).astype(jnp.int32)
    out5 = _embed_lookup(idx3, weight, b, s, d)
    return out5.transpose(2, 4, 0, 1, 3).reshape(b, s, d)
